# baseline (speedup 1.0000x reference)
import functools

import jax
import jax.numpy as jnp
from jax import lax
from jax.experimental import pallas as pl
from jax.experimental.pallas import tpu as pltpu
from jax.experimental.pallas import tpu_sc as plsc

_N = 10000
_E = 320000
_H = 128
_NG = 64
_DEPTH = 4

_NC = 2
_NS = 16
_NW = _NC * _NS
_CH = 128
_KE = -(-_E // (_NW * _CH))
_EPAD = _NW * _CH * _KE
_NPAD = 10240
_STRIPE = _NPAD // _NS

_CHP = 64
_KP = _NPAD // (_NW * _CHP)
_GACC = 128
_GSTRIPE = _GACC // _NS

_mesh = plsc.VectorSubcoreMesh(core_axis_name="c", subcore_axis_name="s")


def _silu(x):
    return x * jax.nn.sigmoid(x)


@functools.partial(
    pl.kernel,
    out_type=jax.ShapeDtypeStruct((_EPAD, 16), jnp.float32),
    mesh=_mesh,
    scratch_types=[
        pltpu.VMEM((_KE, _CH), jnp.int32),
        pltpu.VMEM((_KE, _CH), jnp.int32),
        pltpu.VMEM((_CH, 16), jnp.float32),
        pltpu.VMEM((_CH, 16), jnp.float32),
        pltpu.VMEM((_CH, 16), jnp.float32),
        pltpu.VMEM((_CH, 16), jnp.float32),
        pltpu.VMEM((_CH, 16), jnp.float32),
        pltpu.VMEM((_CH, 16), jnp.float32),
        pltpu.SemaphoreType.DMA,
        pltpu.SemaphoreType.DMA,
        pltpu.SemaphoreType.DMA,
        pltpu.SemaphoreType.DMA,
    ],
    compiler_params=pltpu.CompilerParams(use_tc_tiling_on_sc=False),
)
def _sc_posdiff(pos16, src_i, dst_i, sq, srcs, dsts,
                ab0, ab1, bb0, bb1, ob0, ob1, sg0, sg1, sw0, sw1):
    w = lax.axis_index("s") * _NC + lax.axis_index("c")
    pltpu.sync_copy(src_i.at[w], srcs)
    pltpu.sync_copy(dst_i.at[w], dsts)
    abs_, bbs, obs = (ab0, ab1), (bb0, bb1), (ob0, ob1)
    sgs, sws = (sg0, sg1), (sw0, sw1)

    def base(c):
        return (w * _KE + c) * _CH

    def issue_gather(c, b):
        pltpu.async_copy(pos16.at[srcs.at[c]], abs_[b], sgs[b])
        pltpu.async_copy(pos16.at[dsts.at[c]], bbs[b], sgs[b])

    def wait_gather(c, b):
        pltpu.make_async_copy(pos16.at[srcs.at[c]], abs_[b], sgs[b]).wait()
        pltpu.make_async_copy(pos16.at[dsts.at[c]], bbs[b], sgs[b]).wait()

    def wait_write(c, b):
        pltpu.make_async_copy(obs[b], sq.at[pl.ds(base(c), _CH)],
                              sws[b]).wait()

    def step(c, b):
        wait_gather(c, b)

        @pl.when(c + 1 < _KE)
        def _():
            issue_gather(c + 1, 1 - b)

        @pl.when(c >= 2)
        def _():
            wait_write(c - 2, b)

        ab, bb, ob = abs_[b], bbs[b], obs[b]

        def row(r, _):
            d = ab[r, :] - bb[r, :]
            ob[r, :] = d * d
            return 0

        lax.fori_loop(0, _CH, row, 0)
        pltpu.async_copy(ob, sq.at[pl.ds(base(c), _CH)], sws[b])

    issue_gather(0, 0)

    def body(i, _):
        for b in (0, 1):
            step(2 * i + b, b)
        return 0

    lax.fori_loop(0, _KE // 2, body, 0)
    if _KE % 2:
        step(_KE - 1, (_KE - 1) % 2)
    wait_write(_KE - 2, (_KE - 2) % 2)
    wait_write(_KE - 1, (_KE - 1) % 2)


def _make_sc_gather(n_chunks):
  @functools.partial(
      pl.kernel,
      out_type=jax.ShapeDtypeStruct((_NW * n_chunks * _CH, _H), jnp.float32),
      mesh=_mesh,
      scratch_types=[
          pltpu.VMEM((n_chunks, _CH), jnp.int32),
          pltpu.VMEM((n_chunks, _CH), jnp.int32),
          pltpu.VMEM((_CH, _H), jnp.float32),
          pltpu.VMEM((_CH, _H), jnp.float32),
          pltpu.VMEM((_CH, _H), jnp.float32),
          pltpu.VMEM((_CH, _H), jnp.float32),
          pltpu.VMEM((_CH, _H), jnp.float32),
          pltpu.VMEM((_CH, _H), jnp.float32),
          pltpu.SemaphoreType.DMA,
          pltpu.SemaphoreType.DMA,
          pltpu.SemaphoreType.DMA,
          pltpu.SemaphoreType.DMA,
      ],
  )
  def _sc_gather_add(p_tab, q_tab, src_i, dst_i, t0, srcs, dsts,
                     pb0, pb1, qb0, qb1, ob0, ob1, sg0, sg1, sw0, sw1):
    w = lax.axis_index("s") * _NC + lax.axis_index("c")
    pltpu.sync_copy(src_i.at[w], srcs)
    pltpu.sync_copy(dst_i.at[w], dsts)
    pbs, qbs, obs = (pb0, pb1), (qb0, qb1), (ob0, ob1)
    sgs, sws = (sg0, sg1), (sw0, sw1)

    def base(c):
        return (w * n_chunks + c) * _CH

    def issue_gather(c, b):
        pltpu.async_copy(p_tab.at[dsts.at[c]], pbs[b], sgs[b])
        pltpu.async_copy(q_tab.at[srcs.at[c]], qbs[b], sgs[b])

    def wait_gather(c, b):
        pltpu.make_async_copy(p_tab.at[dsts.at[c]], pbs[b], sgs[b]).wait()
        pltpu.make_async_copy(q_tab.at[srcs.at[c]], qbs[b], sgs[b]).wait()

    def wait_write(c, b):
        pltpu.make_async_copy(obs[b], t0.at[pl.ds(base(c), _CH)],
                              sws[b]).wait()

    def step(c, b):
        wait_gather(c, b)

        @pl.when(c + 1 < n_chunks)
        def _():
            issue_gather(c + 1, 1 - b)

        @pl.when(c >= 2)
        def _():
            wait_write(c - 2, b)

        pb, qb, ob = pbs[b], qbs[b], obs[b]

        def row(r, _):
            for j in range(_H // 16):
                s = pl.ds(j * 16, 16)
                ob[r, s] = pb[r, s] + qb[r, s]
            return 0

        lax.fori_loop(0, _CH, row, 0)
        pltpu.async_copy(ob, t0.at[pl.ds(base(c), _CH)], sws[b])

    issue_gather(0, 0)

    def body(i, _):
        for b in (0, 1):
            step(2 * i + b, b)
        return 0

    lax.fori_loop(0, n_chunks // 2, body, 0)
    if n_chunks % 2:
        step(n_chunks - 1, (n_chunks - 1) % 2)
    wait_write(n_chunks - 2, (n_chunks - 2) % 2)
    wait_write(n_chunks - 1, (n_chunks - 1) % 2)

  return _sc_gather_add


_sc_gather_full = _make_sc_gather(_KE)


def _make_sc_scatter(n_chunks, ch, n_acc):
    stripe = n_acc // _NS

    @functools.partial(
        pl.kernel,
        out_type=jax.ShapeDtypeStruct((_NC, n_acc, _H), jnp.float32),
        mesh=_mesh,
        scratch_types=[
            pltpu.VMEM((n_chunks, ch), jnp.int32),
            pltpu.VMEM((ch, _H), jnp.float32),
            pltpu.VMEM((ch, _H), jnp.float32),
            pltpu.VMEM_SHARED((n_acc, _H), jnp.float32),
            pltpu.SemaphoreType.DMA,
            pltpu.SemaphoreType.DMA,
        ],
    )
    def f(m, idx_i, zrows, out, idxs, mb0, mb1, acc, sl0, sl1):
        cid = lax.axis_index("c")
        sid = lax.axis_index("s")
        w = sid * _NC + cid
        pltpu.sync_copy(idx_i.at[w], idxs)
        pltpu.sync_copy(zrows.at[pl.ds(0, stripe)],
                        acc.at[pl.ds(sid * stripe, stripe)])
        plsc.subcore_barrier()
        mbs, sls = (mb0, mb1), (sl0, sl1)

        def base(c):
            return (w * n_chunks + c) * ch

        def issue_load(c, b):
            pltpu.async_copy(m.at[pl.ds(base(c), ch)], mbs[b], sls[b])

        def wait_load(c, b):
            pltpu.make_async_copy(m.at[pl.ds(base(c), ch)], mbs[b],
                                  sls[b]).wait()

        issue_load(0, 0)

        def chunk2(i, _):
            for b in (0, 1):
                c = 2 * i + b

                @pl.when(c + 1 < n_chunks)
                def _():
                    issue_load(c + 1, 1 - b)

                wait_load(c, b)
                pltpu.sync_copy(mbs[b], acc.at[idxs.at[c]], add=True)
            return 0

        lax.fori_loop(0, n_chunks // 2, chunk2, 0)
        if n_chunks % 2:
            c = n_chunks - 1
            wait_load(c, c % 2)
            pltpu.sync_copy(mbs[c % 2], acc.at[idxs.at[c]], add=True)
        plsc.subcore_barrier()
        pltpu.sync_copy(acc.at[pl.ds(sid * stripe, stripe)],
                        out.at[cid, pl.ds(sid * stripe, stripe)])

    return f


_sc_scatter_edges = _make_sc_scatter(_KE, _CH, _NPAD)


def _full(shape):
    return pl.BlockSpec(shape, lambda i: (0,) * len(shape))


def _mlp2_body(x_ref, w1_ref, b1_ref, w2_ref, b2_ref, o_ref):
    a = _silu(jnp.dot(x_ref[...], w1_ref[...],
                      preferred_element_type=jnp.float32) + b1_ref[...])
    o_ref[...] = jnp.dot(a, w2_ref[...],
                         preferred_element_type=jnp.float32) + b2_ref[...]


def _mlp2(x, w1, b1, w2, b2, bn=1024):
    n = x.shape[0]
    return pl.pallas_call(
        _mlp2_body,
        grid=(n // bn,),
        in_specs=[pl.BlockSpec((bn, _H), lambda i: (i, 0)),
                  _full((_H, _H)), _full((1, _H)),
                  _full((_H, _H)), _full((1, _H))],
        out_specs=pl.BlockSpec((bn, _H), lambda i: (i, 0)),
        out_shape=jax.ShapeDtypeStruct((n, _H), jnp.float32),
    )(x, w1, b1.reshape(1, _H), w2, b2.reshape(1, _H))


def _embed_pq_body(x_ref, w1_ref, b1_ref, w2_ref, b2_ref, wa_ref, wb_ref,
                   h_ref, p_ref, q_ref):
    a = _silu(jnp.dot(x_ref[...], w1_ref[...],
                      preferred_element_type=jnp.float32) + b1_ref[...])
    h = jnp.dot(a, w2_ref[...],
                preferred_element_type=jnp.float32) + b2_ref[...]
    h_ref[...] = h
    p_ref[...] = jnp.dot(h, wa_ref[...], preferred_element_type=jnp.float32)
    q_ref[...] = jnp.dot(h, wb_ref[...], preferred_element_type=jnp.float32)


def _embed_pq(x, w1, b1, w2, b2, wa, wb, bn=1024):
    n = x.shape[0]
    spec = pl.BlockSpec((bn, _H), lambda i: (i, 0))
    shape = jax.ShapeDtypeStruct((n, _H), jnp.float32)
    return pl.pallas_call(
        _embed_pq_body,
        grid=(n // bn,),
        in_specs=[spec, _full((_H, _H)), _full((1, _H)),
                  _full((_H, _H)), _full((1, _H)),
                  _full((_H, _H)), _full((_H, _H))],
        out_specs=[spec, spec, spec],
        out_shape=[shape, shape, shape],
    )(x, w1, b1.reshape(1, _H), w2, b2.reshape(1, _H), wa, wb)


def _d2_body(sq_ref, o_ref):
    o_ref[...] = jnp.sum(sq_ref[...], axis=1, keepdims=True)


def _d2(sq, be=2048):
    return pl.pallas_call(
        _d2_body,
        grid=(_EPAD // be,),
        in_specs=[pl.BlockSpec((be, 16), lambda i: (i, 0))],
        out_specs=pl.BlockSpec((be, 1), lambda i: (i, 0)),
        out_shape=jax.ShapeDtypeStruct((_EPAD, 1), jnp.float32),
    )(sq)


def _edge_body(t_ref, d2_ref, wd_ref, b1_ref, w2_ref, b2_ref, o_ref):
    pre = t_ref[...] + d2_ref[...] * wd_ref[...] + b1_ref[...]
    a = _silu(pre)
    o_ref[...] = _silu(jnp.dot(a, w2_ref[...],
                               preferred_element_type=jnp.float32)
                       + b2_ref[...])


def _edge_mlp(t0, d2, wd, b1, w2, b2, be=2048):
    n = t0.shape[0]
    return pl.pallas_call(
        _edge_body,
        grid=(n // be,),
        in_specs=[pl.BlockSpec((be, _H), lambda i: (i, 0)),
                  pl.BlockSpec((be, 1), lambda i: (i, 0)),
                  _full((1, _H)), _full((1, _H)),
                  _full((_H, _H)), _full((1, _H))],
        out_specs=pl.BlockSpec((be, _H), lambda i: (i, 0)),
        out_shape=jax.ShapeDtypeStruct((n, _H), jnp.float32),
    )(t0, d2, wd.reshape(1, _H), b1.reshape(1, _H), w2, b2.reshape(1, _H))


def _node_body(h_ref, a0_ref, a1_ref, wa_ref, wb_ref, b1_ref, w2_ref, b2_ref,
               o_ref):
    h = h_ref[...]
    agg = a0_ref[...] + a1_ref[...]
    u = _silu(jnp.dot(h, wa_ref[...], preferred_element_type=jnp.float32)
              + jnp.dot(agg, wb_ref[...], preferred_element_type=jnp.float32)
              + b1_ref[...])
    o_ref[...] = h + jnp.dot(u, w2_ref[...],
                             preferred_element_type=jnp.float32) + b2_ref[...]


def _node_update(h, a0, a1, wa, wb, b1, w2, b2, bn=1024):
    n = h.shape[0]
    spec = pl.BlockSpec((bn, _H), lambda i: (i, 0))
    return pl.pallas_call(
        _node_body,
        grid=(n // bn,),
        in_specs=[spec, spec, spec,
                  _full((_H, _H)), _full((_H, _H)), _full((1, _H)),
                  _full((_H, _H)), _full((1, _H))],
        out_specs=spec,
        out_shape=jax.ShapeDtypeStruct((n, _H), jnp.float32),
    )(h, a0, a1, wa, wb, b1.reshape(1, _H), w2, b2.reshape(1, _H))


def _node_pq_body(h_ref, a0_ref, a1_ref, wa_ref, wb_ref, b1_ref, w2_ref,
                  b2_ref, wpa_ref, wpb_ref, h_out, p_ref, q_ref):
    h = h_ref[...]
    agg = a0_ref[...] + a1_ref[...]
    u = _silu(jnp.dot(h, wa_ref[...], preferred_element_type=jnp.float32)
              + jnp.dot(agg, wb_ref[...], preferred_element_type=jnp.float32)
              + b1_ref[...])
    hn = h + jnp.dot(u, w2_ref[...],
                     preferred_element_type=jnp.float32) + b2_ref[...]
    h_out[...] = hn
    p_ref[...] = jnp.dot(hn, wpa_ref[...], preferred_element_type=jnp.float32)
    q_ref[...] = jnp.dot(hn, wpb_ref[...], preferred_element_type=jnp.float32)


def _node_pq(h, a0, a1, wa, wb, b1, w2, b2, wpa, wpb, bn=1024):
    n = h.shape[0]
    spec = pl.BlockSpec((bn, _H), lambda i: (i, 0))
    shape = jax.ShapeDtypeStruct((n, _H), jnp.float32)
    return pl.pallas_call(
        _node_pq_body,
        grid=(n // bn,),
        in_specs=[spec, spec, spec,
                  _full((_H, _H)), _full((_H, _H)), _full((1, _H)),
                  _full((_H, _H)), _full((1, _H)),
                  _full((_H, _H)), _full((_H, _H))],
        out_specs=[spec, spec, spec],
        out_shape=[shape, shape, shape],
    )(h, a0, a1, wa, wb, b1.reshape(1, _H), w2, b2.reshape(1, _H), wpa, wpb)


_PBN = 1024
_PGRID = _NPAD // _PBN


def _pool_head_body(h_ref, bi_ref, w1_ref, b1_ref, w2_ref, b2_ref,
                    wr1_ref, br1_ref, wr2_ref, br2_ref, o_ref, acc_ref):
    i = pl.program_id(0)
    a = _silu(jnp.dot(h_ref[...], w1_ref[...],
                      preferred_element_type=jnp.float32) + b1_ref[...])
    hp = jnp.dot(a, w2_ref[...],
                 preferred_element_type=jnp.float32) + b2_ref[...]
    bi2 = bi_ref[0]
    onehot_t = (bi2 == jax.lax.broadcasted_iota(jnp.int32, (_NG, _PBN), 0)
                ).astype(jnp.float32)
    row = (jax.lax.broadcasted_iota(jnp.int32, (_PBN, 1), 0)
           + i * _PBN)
    hp = jnp.where(row < _N, hp, 0.0)
    pp = jnp.dot(onehot_t, hp, preferred_element_type=jnp.float32)

    @pl.when(i == 0)
    def _():
        acc_ref[...] = pp

    @pl.when(i > 0)
    def _():
        acc_ref[...] = acc_ref[...] + pp

    @pl.when(i == _PGRID - 1)
    def _():
        r = _silu(jnp.dot(acc_ref[...], wr1_ref[...],
                          preferred_element_type=jnp.float32) + br1_ref[...])
        o_ref[...] = jnp.dot(r, wr2_ref[...],
                             preferred_element_type=jnp.float32) + br2_ref[...]


def _pool_head(h, bi3, w1, b1, w2, b2, wr1, br1, wr2, br2):
    return pl.pallas_call(
        _pool_head_body,
        grid=(_PGRID,),
        in_specs=[pl.BlockSpec((_PBN, _H), lambda i: (i, 0)),
                  pl.BlockSpec((1, 1, _PBN), lambda i: (i, 0, 0)),
                  _full((_H, _H)), _full((1, _H)),
                  _full((_H, _H)), _full((1, _H)),
                  _full((_H, _H)), _full((1, _H)),
                  _full((_H, _H)), _full((1, _H))],
        out_specs=_full((_NG, _H)),
        out_shape=jax.ShapeDtypeStruct((_NG, _H), jnp.float32),
        scratch_shapes=[pltpu.VMEM((_NG, _H), jnp.float32)],
    )(h, bi3, w1, b1.reshape(1, _H), w2, b2.reshape(1, _H),
      wr1, br1.reshape(1, _H), wr2, br2.reshape(1, _H))


def kernel(x, pos, edge_index, batch_idx,
           W_e1, b_e1, W_e2, b_e2,
           Wm1, bm1, Wm2, bm2, Wu1, bu1, Wu2, bu2,
           W_p1, b_p1, W_p2, b_p2, W_r1, b_r1, W_r2, b_r2):
    i32 = jnp.int32
    src = edge_index[0].astype(i32)
    dst = edge_index[1].astype(i32)
    pad_e = _EPAD - _E
    src_p = jnp.concatenate([src, jnp.full((pad_e,), _N, i32)])
    dst_p = jnp.concatenate([dst, jnp.full((pad_e,), _N, i32)])
    src_i = src_p.reshape(_NW, _KE, _CH)
    dst_i = dst_p.reshape(_NW, _KE, _CH)
    bat_i = jnp.concatenate(
        [batch_idx.astype(i32), jnp.full((_NPAD - _N,), _NG, i32)]).reshape(
        _PGRID, 1, _PBN)
    x_p = jnp.pad(x, ((0, _NPAD - _N), (0, 0)))
    pos16 = jnp.pad(pos, ((0, _NPAD - _N), (0, 16 - pos.shape[1])))
    zrows = jnp.zeros((_STRIPE, _H), jnp.float32)

    sq = _sc_posdiff(pos16, src_i, dst_i)
    d2 = _d2(sq)
    h, p_tab, q_tab = _embed_pq(x_p, W_e1, b_e1, W_e2, b_e2,
                                Wm1[0, :_H], Wm1[0, _H:2 * _H])

    for l in range(_DEPTH):
        t0 = _sc_gather_full(p_tab, q_tab, src_i, dst_i)
        m = _edge_mlp(t0, d2, Wm1[l, 2 * _H], bm1[l], Wm2[l], bm2[l])
        aggp = _sc_scatter_edges(m, dst_i, zrows)
        if l + 1 < _DEPTH:
            h, p_tab, q_tab = _node_pq(
                h, aggp[0], aggp[1],
                Wu1[l, :_H], Wu1[l, _H:], bu1[l], Wu2[l], bu2[l],
                Wm1[l + 1, :_H], Wm1[l + 1, _H:2 * _H])
        else:
            h = _node_update(h, aggp[0], aggp[1],
                             Wu1[l, :_H], Wu1[l, _H:], bu1[l], Wu2[l],
                             bu2[l])

    return _pool_head(h, bat_i, W_p1, b_p1, W_p2, b_p2,
                      W_r1, b_r1, W_r2, b_r2)


# speedup vs baseline: 1.1061x; 1.1061x over previous
import functools

import jax
import jax.numpy as jnp
from jax import lax
from jax.experimental import pallas as pl
from jax.experimental.pallas import tpu as pltpu
from jax.experimental.pallas import tpu_sc as plsc

_N = 10000
_E = 320000
_H = 128
_NG = 64
_DEPTH = 4

_NC = 2
_NS = 16
_NW = _NC * _NS
_CH = 128
_KE = -(-_E // (_NW * _CH))
_EPAD = _NW * _CH * _KE
_NPAD = 10240
_STRIPE = _NPAD // _NS

_CHP = 64
_KP = _NPAD // (_NW * _CHP)
_GACC = 128
_GSTRIPE = _GACC // _NS

_mesh = plsc.VectorSubcoreMesh(core_axis_name="c", subcore_axis_name="s")


def _silu(x):
    return x * jax.nn.sigmoid(x)


@functools.partial(
    pl.kernel,
    out_type=jax.ShapeDtypeStruct((_EPAD, 16), jnp.float32),
    mesh=_mesh,
    scratch_types=[
        pltpu.VMEM((_KE, _CH), jnp.int32),
        pltpu.VMEM((_KE, _CH), jnp.int32),
        pltpu.VMEM((_CH, 16), jnp.float32),
        pltpu.VMEM((_CH, 16), jnp.float32),
        pltpu.VMEM((_CH, 16), jnp.float32),
        pltpu.VMEM((_CH, 16), jnp.float32),
        pltpu.VMEM((_CH, 16), jnp.float32),
        pltpu.VMEM((_CH, 16), jnp.float32),
        pltpu.SemaphoreType.DMA,
        pltpu.SemaphoreType.DMA,
        pltpu.SemaphoreType.DMA,
        pltpu.SemaphoreType.DMA,
    ],
    compiler_params=pltpu.CompilerParams(use_tc_tiling_on_sc=False),
)
def _sc_posdiff(pos16, src_i, dst_i, sq, srcs, dsts,
                ab0, ab1, bb0, bb1, ob0, ob1, sg0, sg1, sw0, sw1):
    w = lax.axis_index("s") * _NC + lax.axis_index("c")
    pltpu.sync_copy(src_i.at[w], srcs)
    pltpu.sync_copy(dst_i.at[w], dsts)
    abs_, bbs, obs = (ab0, ab1), (bb0, bb1), (ob0, ob1)
    sgs, sws = (sg0, sg1), (sw0, sw1)

    def base(c):
        return (w * _KE + c) * _CH

    def issue_gather(c, b):
        pltpu.async_copy(pos16.at[srcs.at[c]], abs_[b], sgs[b])
        pltpu.async_copy(pos16.at[dsts.at[c]], bbs[b], sgs[b])

    def wait_gather(c, b):
        pltpu.make_async_copy(pos16.at[srcs.at[c]], abs_[b], sgs[b]).wait()
        pltpu.make_async_copy(pos16.at[dsts.at[c]], bbs[b], sgs[b]).wait()

    def wait_write(c, b):
        pltpu.make_async_copy(obs[b], sq.at[pl.ds(base(c), _CH)],
                              sws[b]).wait()

    def step(c, b):
        wait_gather(c, b)

        @pl.when(c + 1 < _KE)
        def _():
            issue_gather(c + 1, 1 - b)

        @pl.when(c >= 2)
        def _():
            wait_write(c - 2, b)

        ab, bb, ob = abs_[b], bbs[b], obs[b]

        def row(r, _):
            d = ab[r, :] - bb[r, :]
            ob[r, :] = d * d
            return 0

        lax.fori_loop(0, _CH, row, 0)
        pltpu.async_copy(ob, sq.at[pl.ds(base(c), _CH)], sws[b])

    issue_gather(0, 0)

    def body(i, _):
        for b in (0, 1):
            step(2 * i + b, b)
        return 0

    lax.fori_loop(0, _KE // 2, body, 0)
    if _KE % 2:
        step(_KE - 1, (_KE - 1) % 2)
    wait_write(_KE - 2, (_KE - 2) % 2)
    wait_write(_KE - 1, (_KE - 1) % 2)


def _make_sc_gather(n_chunks):
  @functools.partial(
      pl.kernel,
      out_type=jax.ShapeDtypeStruct((_NW * n_chunks * _CH, _H), jnp.float32),
      mesh=_mesh,
      scratch_types=[
          pltpu.VMEM((n_chunks, _CH), jnp.int32),
          pltpu.VMEM((n_chunks, _CH), jnp.int32),
          pltpu.VMEM((_CH, _H), jnp.float32),
          pltpu.VMEM((_CH, _H), jnp.float32),
          pltpu.VMEM((_CH, _H), jnp.float32),
          pltpu.VMEM((_CH, _H), jnp.float32),
          pltpu.VMEM((_CH, _H), jnp.float32),
          pltpu.VMEM((_CH, _H), jnp.float32),
          pltpu.SemaphoreType.DMA,
          pltpu.SemaphoreType.DMA,
          pltpu.SemaphoreType.DMA,
          pltpu.SemaphoreType.DMA,
      ],
  )
  def _sc_gather_add(p_tab, q_tab, src_i, dst_i, t0, srcs, dsts,
                     pb0, pb1, qb0, qb1, ob0, ob1, sg0, sg1, sw0, sw1):
    w = lax.axis_index("s") * _NC + lax.axis_index("c")
    pltpu.sync_copy(src_i.at[w], srcs)
    pltpu.sync_copy(dst_i.at[w], dsts)
    pbs, qbs, obs = (pb0, pb1), (qb0, qb1), (ob0, ob1)
    sgs, sws = (sg0, sg1), (sw0, sw1)

    def base(c):
        return (w * n_chunks + c) * _CH

    def issue_gather(c, b):
        pltpu.async_copy(p_tab.at[dsts.at[c]], pbs[b], sgs[b])
        pltpu.async_copy(q_tab.at[srcs.at[c]], qbs[b], sgs[b])

    def wait_gather(c, b):
        pltpu.make_async_copy(p_tab.at[dsts.at[c]], pbs[b], sgs[b]).wait()
        pltpu.make_async_copy(q_tab.at[srcs.at[c]], qbs[b], sgs[b]).wait()

    def wait_write(c, b):
        pltpu.make_async_copy(obs[b], t0.at[pl.ds(base(c), _CH)],
                              sws[b]).wait()

    def step(c, b):
        wait_gather(c, b)

        @pl.when(c + 1 < n_chunks)
        def _():
            issue_gather(c + 1, 1 - b)

        @pl.when(c >= 2)
        def _():
            wait_write(c - 2, b)

        pb, qb, ob = pbs[b], qbs[b], obs[b]

        def row(r, _):
            for j in range(_H // 16):
                s = pl.ds(j * 16, 16)
                ob[r, s] = pb[r, s] + qb[r, s]
            return 0

        lax.fori_loop(0, _CH, row, 0)
        pltpu.async_copy(ob, t0.at[pl.ds(base(c), _CH)], sws[b])

    issue_gather(0, 0)

    def body(i, _):
        for b in (0, 1):
            step(2 * i + b, b)
        return 0

    lax.fori_loop(0, n_chunks // 2, body, 0)
    if n_chunks % 2:
        step(n_chunks - 1, (n_chunks - 1) % 2)
    wait_write(n_chunks - 2, (n_chunks - 2) % 2)
    wait_write(n_chunks - 1, (n_chunks - 1) % 2)

  return _sc_gather_add


_sc_gather_full = _make_sc_gather(_KE)


def _make_sc_scatter(n_chunks, ch, n_acc):
    stripe = n_acc // _NS

    @functools.partial(
        pl.kernel,
        out_type=jax.ShapeDtypeStruct((_NC, n_acc, _H), jnp.float32),
        mesh=_mesh,
        scratch_types=[
            pltpu.VMEM((n_chunks, ch), jnp.int32),
            pltpu.VMEM((ch, _H), jnp.float32),
            pltpu.VMEM((ch, _H), jnp.float32),
            pltpu.VMEM_SHARED((n_acc, _H), jnp.float32),
            pltpu.SemaphoreType.DMA,
            pltpu.SemaphoreType.DMA,
        ],
    )
    def f(m, idx_i, zrows, out, idxs, mb0, mb1, acc, sl0, sl1):
        cid = lax.axis_index("c")
        sid = lax.axis_index("s")
        w = sid * _NC + cid
        pltpu.sync_copy(idx_i.at[w], idxs)
        pltpu.sync_copy(zrows.at[pl.ds(0, stripe)],
                        acc.at[pl.ds(sid * stripe, stripe)])
        plsc.subcore_barrier()
        mbs, sls = (mb0, mb1), (sl0, sl1)

        def base(c):
            return (w * n_chunks + c) * ch

        def issue_load(c, b):
            pltpu.async_copy(m.at[pl.ds(base(c), ch)], mbs[b], sls[b])

        def wait_load(c, b):
            pltpu.make_async_copy(m.at[pl.ds(base(c), ch)], mbs[b],
                                  sls[b]).wait()

        issue_load(0, 0)

        def chunk2(i, _):
            for b in (0, 1):
                c = 2 * i + b

                @pl.when(c + 1 < n_chunks)
                def _():
                    issue_load(c + 1, 1 - b)

                wait_load(c, b)
                pltpu.sync_copy(mbs[b], acc.at[idxs.at[c]], add=True)
            return 0

        lax.fori_loop(0, n_chunks // 2, chunk2, 0)
        if n_chunks % 2:
            c = n_chunks - 1
            wait_load(c, c % 2)
            pltpu.sync_copy(mbs[c % 2], acc.at[idxs.at[c]], add=True)
        plsc.subcore_barrier()
        pltpu.sync_copy(acc.at[pl.ds(sid * stripe, stripe)],
                        out.at[cid, pl.ds(sid * stripe, stripe)])

    return f


_sc_scatter_edges = _make_sc_scatter(_KE, _CH, _NPAD)


def _full(shape):
    return pl.BlockSpec(shape, lambda i: (0,) * len(shape))


def _mlp2_body(x_ref, w1_ref, b1_ref, w2_ref, b2_ref, o_ref):
    a = _silu(jnp.dot(x_ref[...], w1_ref[...],
                      preferred_element_type=jnp.float32) + b1_ref[...])
    o_ref[...] = jnp.dot(a, w2_ref[...],
                         preferred_element_type=jnp.float32) + b2_ref[...]


def _mlp2(x, w1, b1, w2, b2, bn=1024):
    n = x.shape[0]
    return pl.pallas_call(
        _mlp2_body,
        grid=(n // bn,),
        in_specs=[pl.BlockSpec((bn, _H), lambda i: (i, 0)),
                  _full((_H, _H)), _full((1, _H)),
                  _full((_H, _H)), _full((1, _H))],
        out_specs=pl.BlockSpec((bn, _H), lambda i: (i, 0)),
        out_shape=jax.ShapeDtypeStruct((n, _H), jnp.float32),
    )(x, w1, b1.reshape(1, _H), w2, b2.reshape(1, _H))


def _pq_body(h_ref, a_ref, b_ref, p_ref, q_ref):
    h = h_ref[...]
    p_ref[...] = jnp.dot(h, a_ref[...], preferred_element_type=jnp.float32)
    q_ref[...] = jnp.dot(h, b_ref[...], preferred_element_type=jnp.float32)


def _pq(h, a, b, bn=1024):
    n = h.shape[0]
    spec = pl.BlockSpec((bn, _H), lambda i: (i, 0))
    shape = jax.ShapeDtypeStruct((n, _H), jnp.float32)
    return pl.pallas_call(
        _pq_body,
        grid=(n // bn,),
        in_specs=[spec, _full((_H, _H)), _full((_H, _H))],
        out_specs=[spec, spec],
        out_shape=[shape, shape],
    )(h, a, b)


def _d2_body(sq_ref, o_ref):
    o_ref[...] = jnp.sum(sq_ref[...], axis=1, keepdims=True)


def _d2(sq, be=2048):
    return pl.pallas_call(
        _d2_body,
        grid=(_EPAD // be,),
        in_specs=[pl.BlockSpec((be, 16), lambda i: (i, 0))],
        out_specs=pl.BlockSpec((be, 1), lambda i: (i, 0)),
        out_shape=jax.ShapeDtypeStruct((_EPAD, 1), jnp.float32),
    )(sq)


def _edge_body(t_ref, d2_ref, wd_ref, b1_ref, w2_ref, b2_ref, o_ref):
    pre = t_ref[...] + d2_ref[...] * wd_ref[...] + b1_ref[...]
    a = _silu(pre)
    o_ref[...] = _silu(jnp.dot(a, w2_ref[...],
                               preferred_element_type=jnp.float32)
                       + b2_ref[...])


def _edge_mlp(t0, d2, wd, b1, w2, b2, be=2048):
    n = t0.shape[0]
    return pl.pallas_call(
        _edge_body,
        grid=(n // be,),
        in_specs=[pl.BlockSpec((be, _H), lambda i: (i, 0)),
                  pl.BlockSpec((be, 1), lambda i: (i, 0)),
                  _full((1, _H)), _full((1, _H)),
                  _full((_H, _H)), _full((1, _H))],
        out_specs=pl.BlockSpec((be, _H), lambda i: (i, 0)),
        out_shape=jax.ShapeDtypeStruct((n, _H), jnp.float32),
    )(t0, d2, wd.reshape(1, _H), b1.reshape(1, _H), w2, b2.reshape(1, _H))


def _node_body(h_ref, a0_ref, a1_ref, wa_ref, wb_ref, b1_ref, w2_ref, b2_ref,
               o_ref):
    h = h_ref[...]
    agg = a0_ref[...] + a1_ref[...]
    u = _silu(jnp.dot(h, wa_ref[...], preferred_element_type=jnp.float32)
              + jnp.dot(agg, wb_ref[...], preferred_element_type=jnp.float32)
              + b1_ref[...])
    o_ref[...] = h + jnp.dot(u, w2_ref[...],
                             preferred_element_type=jnp.float32) + b2_ref[...]


def _node_update(h, a0, a1, wa, wb, b1, w2, b2, bn=1024):
    n = h.shape[0]
    spec = pl.BlockSpec((bn, _H), lambda i: (i, 0))
    return pl.pallas_call(
        _node_body,
        grid=(n // bn,),
        in_specs=[spec, spec, spec,
                  _full((_H, _H)), _full((_H, _H)), _full((1, _H)),
                  _full((_H, _H)), _full((1, _H))],
        out_specs=spec,
        out_shape=jax.ShapeDtypeStruct((n, _H), jnp.float32),
    )(h, a0, a1, wa, wb, b1.reshape(1, _H), w2, b2.reshape(1, _H))


_PBN = 1024
_PGRID = _NPAD // _PBN


def _pool_head_body(h_ref, bi_ref, w1_ref, b1_ref, w2_ref, b2_ref,
                    wr1_ref, br1_ref, wr2_ref, br2_ref, o_ref, acc_ref):
    i = pl.program_id(0)
    a = _silu(jnp.dot(h_ref[...], w1_ref[...],
                      preferred_element_type=jnp.float32) + b1_ref[...])
    hp = jnp.dot(a, w2_ref[...],
                 preferred_element_type=jnp.float32) + b2_ref[...]
    bi2 = bi_ref[0]
    onehot_t = (bi2 == jax.lax.broadcasted_iota(jnp.int32, (_NG, _PBN), 0)
                ).astype(jnp.float32)
    row = (jax.lax.broadcasted_iota(jnp.int32, (_PBN, 1), 0)
           + i * _PBN)
    hp = jnp.where(row < _N, hp, 0.0)
    pp = jnp.dot(onehot_t, hp, preferred_element_type=jnp.float32)

    @pl.when(i == 0)
    def _():
        acc_ref[...] = pp

    @pl.when(i > 0)
    def _():
        acc_ref[...] = acc_ref[...] + pp

    @pl.when(i == _PGRID - 1)
    def _():
        r = _silu(jnp.dot(acc_ref[...], wr1_ref[...],
                          preferred_element_type=jnp.float32) + br1_ref[...])
        o_ref[...] = jnp.dot(r, wr2_ref[...],
                             preferred_element_type=jnp.float32) + br2_ref[...]


def _pool_head(h, bi3, w1, b1, w2, b2, wr1, br1, wr2, br2):
    return pl.pallas_call(
        _pool_head_body,
        grid=(_PGRID,),
        in_specs=[pl.BlockSpec((_PBN, _H), lambda i: (i, 0)),
                  pl.BlockSpec((1, 1, _PBN), lambda i: (i, 0, 0)),
                  _full((_H, _H)), _full((1, _H)),
                  _full((_H, _H)), _full((1, _H)),
                  _full((_H, _H)), _full((1, _H)),
                  _full((_H, _H)), _full((1, _H))],
        out_specs=_full((_NG, _H)),
        out_shape=jax.ShapeDtypeStruct((_NG, _H), jnp.float32),
        scratch_shapes=[pltpu.VMEM((_NG, _H), jnp.float32)],
    )(h, bi3, w1, b1.reshape(1, _H), w2, b2.reshape(1, _H),
      wr1, br1.reshape(1, _H), wr2, br2.reshape(1, _H))


def kernel(x, pos, edge_index, batch_idx,
           W_e1, b_e1, W_e2, b_e2,
           Wm1, bm1, Wm2, bm2, Wu1, bu1, Wu2, bu2,
           W_p1, b_p1, W_p2, b_p2, W_r1, b_r1, W_r2, b_r2):
    i32 = jnp.int32
    src = edge_index[0].astype(i32)
    dst = edge_index[1].astype(i32)
    pad_e = _EPAD - _E
    src_p = jnp.concatenate([src, jnp.full((pad_e,), _N, i32)])
    dst_p = jnp.concatenate([dst, jnp.full((pad_e,), _N, i32)])
    src_i = src_p.reshape(_NW, _KE, _CH)
    dst_i = dst_p.reshape(_NW, _KE, _CH)
    bat_i = jnp.concatenate(
        [batch_idx.astype(i32), jnp.full((_NPAD - _N,), _NG, i32)]).reshape(
        _PGRID, 1, _PBN)
    x_p = jnp.pad(x, ((0, _NPAD - _N), (0, 0)))
    pos16 = jnp.pad(pos, ((0, _NPAD - _N), (0, 16 - pos.shape[1])))
    zrows = jnp.zeros((_STRIPE, _H), jnp.float32)

    sq = _sc_posdiff(pos16, src_i, dst_i)
    d2 = _d2(sq)
    h = _mlp2(x_p, W_e1, b_e1, W_e2, b_e2)

    for l in range(_DEPTH):
        p_tab, q_tab = _pq(h, Wm1[l, :_H], Wm1[l, _H:2 * _H])
        t0 = _sc_gather_full(p_tab, q_tab, src_i, dst_i)
        m = _edge_mlp(t0, d2, Wm1[l, 2 * _H], bm1[l], Wm2[l], bm2[l])
        aggp = _sc_scatter_edges(m, dst_i, zrows)
        h = _node_update(h, aggp[0], aggp[1],
                         Wu1[l, :_H], Wu1[l, _H:], bu1[l], Wu2[l], bu2[l])

    return _pool_head(h, bat_i, W_p1, b_p1, W_p2, b_p2,
                      W_r1, b_r1, W_r2, b_r2)
